# baseline (speedup 1.0000x reference)
import jax
import jax.numpy as jnp
from jax.experimental import pallas as pl

B, N, DIN, H, DOUT = 16, 512, 128, 64, 64


def _gcn_fused_kernel(a_ref, x_ref, w1_ref, b1_ref, w2_ref, b2_ref, out_ref):
    A = a_ref[0]
    x = x_ref[0]

    ones = jnp.ones((N, 1), dtype=jnp.float32)
    deg = jax.lax.dot_general(A, ones, (((0,), (0,)), ((), ())),
                              preferred_element_type=jnp.float32)
    dinv = jnp.where(deg > 0, jax.lax.rsqrt(deg), 0.0)

    xw = jnp.dot(x, w1_ref[...], preferred_element_type=jnp.float32)
    h = jax.lax.dot_general(A, xw * dinv, (((0,), (0,)), ((), ())),
                            preferred_element_type=jnp.float32)
    h = jnp.maximum(h * dinv + b1_ref[...][None, :], 0.0)

    hw = jnp.dot(h, w2_ref[...], preferred_element_type=jnp.float32)
    o = jax.lax.dot_general(A, hw * dinv, (((0,), (0,)), ((), ())),
                            preferred_element_type=jnp.float32)
    out_ref[0] = jnp.maximum(o * dinv + b2_ref[...][None, :], 0.0)


def kernel(edge_features, edge_weights, W1, b1, W2, b2):
    return pl.pallas_call(
        _gcn_fused_kernel,
        grid=(B,),
        in_specs=[
            pl.BlockSpec((1, N, N), lambda b: (b, 0, 0)),
            pl.BlockSpec((1, N, DIN), lambda b: (b, 0, 0)),
            pl.BlockSpec((DIN, H), lambda b: (0, 0)),
            pl.BlockSpec((H,), lambda b: (0,)),
            pl.BlockSpec((H, DOUT), lambda b: (0, 0)),
            pl.BlockSpec((DOUT,), lambda b: (0,)),
        ],
        out_specs=pl.BlockSpec((1, N, DOUT), lambda b: (b, 0, 0)),
        out_shape=jax.ShapeDtypeStruct((B, N, DOUT), jnp.float32),
    )(edge_weights, edge_features, W1, b1, W2, b2)


# speedup vs baseline: 1.1690x; 1.1690x over previous
import jax
import jax.numpy as jnp
from jax.experimental import pallas as pl

B, N, DIN, H, DOUT = 16, 512, 128, 64, 64


def _gcn_fused_kernel(a_ref, x_ref, w1_ref, b1_ref, w2_ref, b2_ref, out_ref):
    A = a_ref[0]
    x = x_ref[0]

    deg = jnp.sum(A, axis=0, keepdims=True)
    dinv = jnp.where(deg > 0, jax.lax.rsqrt(deg), 0.0)

    xwT = jax.lax.dot_general(w1_ref[...], x, (((0,), (1,)), ((), ())),
                              preferred_element_type=jnp.float32)

    t1 = jnp.dot(xwT * dinv, A, preferred_element_type=jnp.float32)
    h1 = jnp.maximum(t1 * dinv + b1_ref[...][:, None], 0.0)

    hwT = jax.lax.dot_general(w2_ref[...], h1, (((0,), (0,)), ((), ())),
                              preferred_element_type=jnp.float32)
    t2 = jnp.dot(hwT * dinv, A, preferred_element_type=jnp.float32)
    o2 = jnp.maximum(t2 * dinv + b2_ref[...][:, None], 0.0)

    out_ref[0] = o2.T


def kernel(edge_features, edge_weights, W1, b1, W2, b2):
    return pl.pallas_call(
        _gcn_fused_kernel,
        grid=(B,),
        in_specs=[
            pl.BlockSpec((1, N, N), lambda b: (b, 0, 0)),
            pl.BlockSpec((1, N, DIN), lambda b: (b, 0, 0)),
            pl.BlockSpec((DIN, H), lambda b: (0, 0)),
            pl.BlockSpec((H,), lambda b: (0,)),
            pl.BlockSpec((H, DOUT), lambda b: (0, 0)),
            pl.BlockSpec((DOUT,), lambda b: (0,)),
        ],
        out_specs=pl.BlockSpec((1, N, DOUT), lambda b: (b, 0, 0)),
        out_shape=jax.ShapeDtypeStruct((B, N, DOUT), jnp.float32),
    )(edge_weights, edge_features, W1, b1, W2, b2)
